# Initial kernel scaffold; baseline (speedup 1.0000x reference)
import jax
import jax.numpy as jnp
from jax.experimental import pallas as pl


def kernel(pcd):
    raise NotImplementedError("write your pallas kernel here")


# speedup vs baseline: 36.0043x; 36.0043x over previous
import jax
import jax.numpy as jnp
from jax.experimental import pallas as pl
from jax.experimental.pallas import tpu as pltpu

_H = 0.0005
_B = 16
_N = 2048
_R = 256
_NT = _N // _R


def _loss_kernel(xr_ref, xa_ref, out_ref):
    b = pl.program_id(0)
    j = pl.program_id(1)

    xr = xr_ref[0]
    xa = xa_ref[0]
    sq_r = jnp.sum(xr * xr, axis=0).reshape(_R, 1)
    sq_a = jnp.sum(xa * xa, axis=0).reshape(1, _N)
    dot = jax.lax.dot_general(
        xr, xa, (((0,), (0,)), ((), ())),
        preferred_element_type=jnp.float32,
    )
    d2 = jnp.maximum(sq_r + sq_a - 2.0 * dot, 0.0)

    t = jnp.zeros((_R, 1), jnp.float32)
    acc = jnp.zeros((_R, 1), jnp.float32)
    m = jnp.min(d2, axis=1, keepdims=True)
    for it in range(5):
        eq = d2 == m
        e = jnp.sum(eq.astype(jnp.float32), axis=1, keepdims=True)
        ov = jnp.maximum(
            jnp.minimum(t + e, 5.0) - jnp.maximum(t, 1.0), 0.0
        )
        g = jnp.maximum(_H - m * m, 0.0)
        acc = acc + ov * g
        t = t + e
        if it < 4:
            d2 = jnp.where(eq, jnp.inf, d2)
            m = jnp.min(d2, axis=1, keepdims=True)

    s = jnp.sum(acc).reshape(1, 1)

    @pl.when(jnp.logical_and(b == 0, j == 0))
    def _():
        out_ref[:, :] = jnp.zeros((1, 1), jnp.float32)

    out_ref[:, :] += s

    @pl.when(jnp.logical_and(b == _B - 1, j == _NT - 1))
    def _():
        out_ref[:, :] *= 1.0 / (_B * _N * 4)


def kernel(pcd):
    out = pl.pallas_call(
        _loss_kernel,
        grid=(_B, _NT),
        in_specs=[
            pl.BlockSpec((1, 3, _R), lambda b, j: (b, 0, j)),
            pl.BlockSpec((1, 3, _N), lambda b, j: (b, 0, 0)),
        ],
        out_specs=pl.BlockSpec((1, 1), lambda b, j: (0, 0)),
        out_shape=jax.ShapeDtypeStruct((1, 1), jnp.float32),
    )(pcd, pcd)
    return out[0, 0]
